# Initial kernel scaffold; baseline (speedup 1.0000x reference)
import jax
import jax.numpy as jnp
from jax.experimental import pallas as pl


def kernel(x, Wq, Wk, Wv, Wo, Wr, W1, b1, W2, b2):
    raise NotImplementedError("write your pallas kernel here")


# speedup vs baseline: 1.0392x; 1.0392x over previous
import functools

import jax
import jax.numpy as jnp
from jax.experimental import pallas as pl

B, S, D, H, DH = 1, 2048, 1024, 16, 64
E, K, DFF = 8, 2, 4096

BQ = 512
BS = 512
FB = 1024


def _qkv_body(x_ref, wq_ref, wk_ref, wv_ref, q_ref, k_ref, v_ref):
    x = x_ref[...]
    q_ref[0] = jnp.dot(x, wq_ref[0], preferred_element_type=jnp.float32)
    k_ref[0] = jnp.dot(x, wk_ref[0], preferred_element_type=jnp.float32)
    v_ref[0] = jnp.dot(x, wv_ref[0], preferred_element_type=jnp.float32)


def _attn_body(q_ref, k_ref, v_ref, o_ref):
    q = q_ref[0]
    k = k_ref[0]
    v = v_ref[0]
    s = jnp.dot(q, k.T, preferred_element_type=jnp.float32) * (1.0 / (DH ** 0.5))
    s = s - jnp.max(s, axis=-1, keepdims=True)
    p = jnp.exp(s)
    p = p / jnp.sum(p, axis=-1, keepdims=True)
    o_ref[0] = jnp.dot(p, v, preferred_element_type=jnp.float32)


def _proj_router_body(o_ref, x_ref, wo_ref, wr_ref, x1_ref, gate_ref):
    x1 = jnp.dot(o_ref[...], wo_ref[...], preferred_element_type=jnp.float32) + x_ref[...]
    x1_ref[...] = x1
    logits = jnp.dot(x1, wr_ref[...], preferred_element_type=jnp.float32)
    m = jnp.max(logits, axis=-1, keepdims=True)
    p = jnp.exp(logits - m)
    probs = p / jnp.sum(p, axis=-1, keepdims=True)
    lane = jax.lax.broadcasted_iota(jnp.int32, probs.shape, 1)
    v0 = jnp.max(probs, axis=-1, keepdims=True)
    i0 = jnp.min(jnp.where(probs == v0, lane, E), axis=-1, keepdims=True)
    probs1 = jnp.where(lane == i0, -jnp.inf, probs)
    v1 = jnp.max(probs1, axis=-1, keepdims=True)
    i1 = jnp.min(jnp.where(probs1 == v1, lane, E), axis=-1, keepdims=True)
    denom = v0 + v1 + 1e-9
    w0 = v0 / denom
    w1 = v1 / denom
    gate_ref[...] = jnp.where(lane == i0, w0, 0.0) + jnp.where(lane == i1, w1, 0.0)


def _moe_body(x1_ref, gate_ref, w1_ref, b1_ref, w2_ref, b2_ref, out_ref):
    e = pl.program_id(1)
    f = pl.program_id(2)
    x1 = x1_ref[...]
    lane = jax.lax.broadcasted_iota(jnp.int32, gate_ref.shape, 1)
    g = jnp.sum(jnp.where(lane == e, gate_ref[...], 0.0), axis=-1,
                keepdims=True)
    h = jnp.maximum(jnp.dot(x1, w1_ref[0], preferred_element_type=jnp.float32)
                    + b1_ref[0, 0], 0.0)
    acc = jnp.dot(h, w2_ref[0], preferred_element_type=jnp.float32)

    @pl.when(jnp.logical_and(e == 0, f == 0))
    def _():
        out_ref[...] = x1

    @pl.when(f == 0)
    def _():
        out_ref[...] += g * b2_ref[0, 0]

    out_ref[...] += g * acc


def kernel(x, Wq, Wk, Wv, Wo, Wr, W1, b1, W2, b2):
    xf = x.reshape(S, D)
    wq_h = Wq.reshape(D, H, DH).transpose(1, 0, 2)
    wk_h = Wk.reshape(D, H, DH).transpose(1, 0, 2)
    wv_h = Wv.reshape(D, H, DH).transpose(1, 0, 2)
    b1_3 = b1.reshape(E, 1, DFF)
    b2_3 = b2.reshape(E, 1, D)

    q, k, v = pl.pallas_call(
        _qkv_body,
        grid=(H,),
        in_specs=[
            pl.BlockSpec((S, D), lambda h: (0, 0)),
            pl.BlockSpec((1, D, DH), lambda h: (h, 0, 0)),
            pl.BlockSpec((1, D, DH), lambda h: (h, 0, 0)),
            pl.BlockSpec((1, D, DH), lambda h: (h, 0, 0)),
        ],
        out_specs=[
            pl.BlockSpec((1, S, DH), lambda h: (h, 0, 0)),
            pl.BlockSpec((1, S, DH), lambda h: (h, 0, 0)),
            pl.BlockSpec((1, S, DH), lambda h: (h, 0, 0)),
        ],
        out_shape=[jax.ShapeDtypeStruct((H, S, DH), jnp.float32)] * 3,
    )(xf, wq_h, wk_h, wv_h)

    o_h = pl.pallas_call(
        _attn_body,
        grid=(H, S // BQ),
        in_specs=[
            pl.BlockSpec((1, BQ, DH), lambda h, s: (h, s, 0)),
            pl.BlockSpec((1, S, DH), lambda h, s: (h, 0, 0)),
            pl.BlockSpec((1, S, DH), lambda h, s: (h, 0, 0)),
        ],
        out_specs=pl.BlockSpec((1, BQ, DH), lambda h, s: (h, s, 0)),
        out_shape=jax.ShapeDtypeStruct((H, S, DH), jnp.float32),
    )(q, k, v)
    o = o_h.transpose(1, 0, 2).reshape(S, D)

    x1, gate = pl.pallas_call(
        _proj_router_body,
        grid=(S // BS,),
        in_specs=[
            pl.BlockSpec((BS, D), lambda s: (s, 0)),
            pl.BlockSpec((BS, D), lambda s: (s, 0)),
            pl.BlockSpec((D, D), lambda s: (0, 0)),
            pl.BlockSpec((D, E), lambda s: (0, 0)),
        ],
        out_specs=[
            pl.BlockSpec((BS, D), lambda s: (s, 0)),
            pl.BlockSpec((BS, E), lambda s: (s, 0)),
        ],
        out_shape=[
            jax.ShapeDtypeStruct((S, D), jnp.float32),
            jax.ShapeDtypeStruct((S, E), jnp.float32),
        ],
    )(o, xf, Wo, Wr)

    out = pl.pallas_call(
        _moe_body,
        grid=(S // BS, E, DFF // FB),
        in_specs=[
            pl.BlockSpec((BS, D), lambda s, e, f: (s, 0)),
            pl.BlockSpec((BS, E), lambda s, e, f: (s, 0)),
            pl.BlockSpec((1, D, FB), lambda s, e, f: (e, 0, f)),
            pl.BlockSpec((1, 1, FB), lambda s, e, f: (e, 0, f)),
            pl.BlockSpec((1, FB, D), lambda s, e, f: (e, f, 0)),
            pl.BlockSpec((1, 1, D), lambda s, e, f: (e, 0, 0)),
        ],
        out_specs=pl.BlockSpec((BS, D), lambda s, e, f: (s, 0)),
        out_shape=jax.ShapeDtypeStruct((S, D), jnp.float32),
    )(x1, gate, W1, b1_3, W2, b2_3)

    return out.reshape(B, S, D)
